# baseline (speedup 1.0000x reference)
import jax
import jax.numpy as jnp
from jax.experimental import pallas as pl

_NB_SEQ_LEN = 2048
_D = 1024
_BATCH = 4
_BS = 128
_EPS = 1e-5


def _embed_ln_kernel(x_ref, pos_ref, w_ref, b_ref, out_ref):
    h = x_ref[...] + pos_ref[...][None, :, :]
    mu = jnp.mean(h, axis=-1, keepdims=True)
    d = h - mu
    var = jnp.mean(d * d, axis=-1, keepdims=True)
    out_ref[...] = d * jax.lax.rsqrt(var + _EPS) * w_ref[...] + b_ref[...]


def kernel(x, pos_embed, ln_w, ln_b, batch_size_unused):
    del batch_size_unused
    w2 = ln_w.reshape(1, _D)
    b2 = ln_b.reshape(1, _D)
    grid = (_NB_SEQ_LEN // _BS,)
    return pl.pallas_call(
        _embed_ln_kernel,
        grid=grid,
        in_specs=[
            pl.BlockSpec((_BATCH, _BS, _D), lambda s: (0, s, 0)),
            pl.BlockSpec((_BS, _D), lambda s: (s, 0)),
            pl.BlockSpec((1, _D), lambda s: (0, 0)),
            pl.BlockSpec((1, _D), lambda s: (0, 0)),
        ],
        out_specs=pl.BlockSpec((_BATCH, _BS, _D), lambda s: (0, s, 0)),
        out_shape=jax.ShapeDtypeStruct((_BATCH, _NB_SEQ_LEN, _D), jnp.float32),
    )(x, pos_embed, w2, b2)


# speedup vs baseline: 1.0953x; 1.0953x over previous
import jax
import jax.numpy as jnp
from jax.experimental import pallas as pl

_NB_SEQ_LEN = 2048
_D = 1024
_BATCH = 4
_BS = 512
_EPS = 1e-5


def _embed_ln_kernel(x_ref, pos_ref, w_ref, b_ref, out_ref):
    h = x_ref[...] + pos_ref[...][None, :, :]
    mu = jnp.mean(h, axis=-1, keepdims=True)
    d = h - mu
    var = jnp.mean(d * d, axis=-1, keepdims=True)
    out_ref[...] = d * jax.lax.rsqrt(var + _EPS) * w_ref[...] + b_ref[...]


def kernel(x, pos_embed, ln_w, ln_b, batch_size_unused):
    del batch_size_unused
    w2 = ln_w.reshape(1, _D)
    b2 = ln_b.reshape(1, _D)
    grid = (_NB_SEQ_LEN // _BS,)
    return pl.pallas_call(
        _embed_ln_kernel,
        grid=grid,
        in_specs=[
            pl.BlockSpec((_BATCH, _BS, _D), lambda s: (0, s, 0)),
            pl.BlockSpec((_BS, _D), lambda s: (s, 0)),
            pl.BlockSpec((1, _D), lambda s: (0, 0)),
            pl.BlockSpec((1, _D), lambda s: (0, 0)),
        ],
        out_specs=pl.BlockSpec((_BATCH, _BS, _D), lambda s: (0, s, 0)),
        out_shape=jax.ShapeDtypeStruct((_BATCH, _NB_SEQ_LEN, _D), jnp.float32),
    )(x, pos_embed, w2, b2)


# speedup vs baseline: 1.1034x; 1.0074x over previous
import jax
import jax.numpy as jnp
from jax.experimental import pallas as pl

_NB_SEQ_LEN = 2048
_D = 1024
_BATCH = 4
_BS = 256
_EPS = 1e-5


def _embed_ln_kernel(x_ref, pos_ref, w_ref, b_ref, out_ref):
    h = x_ref[...] + pos_ref[...][None, :, :]
    mu = jnp.mean(h, axis=-1, keepdims=True)
    d = h - mu
    var = jnp.mean(d * d, axis=-1, keepdims=True)
    out_ref[...] = d * jax.lax.rsqrt(var + _EPS) * w_ref[...] + b_ref[...]


def kernel(x, pos_embed, ln_w, ln_b, batch_size_unused):
    del batch_size_unused
    w2 = ln_w.reshape(1, _D)
    b2 = ln_b.reshape(1, _D)
    grid = (_NB_SEQ_LEN // _BS,)
    return pl.pallas_call(
        _embed_ln_kernel,
        grid=grid,
        in_specs=[
            pl.BlockSpec((_BATCH, _BS, _D), lambda s: (0, s, 0)),
            pl.BlockSpec((_BS, _D), lambda s: (s, 0)),
            pl.BlockSpec((1, _D), lambda s: (0, 0)),
            pl.BlockSpec((1, _D), lambda s: (0, 0)),
        ],
        out_specs=pl.BlockSpec((_BATCH, _BS, _D), lambda s: (0, s, 0)),
        out_shape=jax.ShapeDtypeStruct((_BATCH, _NB_SEQ_LEN, _D), jnp.float32),
    )(x, pos_embed, w2, b2)
